# baseline (speedup 1.0000x reference)
import jax
import jax.numpy as jnp
from jax.experimental import pallas as pl


ROWS_PER_BLOCK = 512


def _copy_block(x_ref, o_ref):
    o_ref[...] = x_ref[...]


def kernel(x):
    n, d = x.shape
    grid = (n // ROWS_PER_BLOCK,)
    gathered = pl.pallas_call(
        _copy_block,
        grid=grid,
        in_specs=[pl.BlockSpec((ROWS_PER_BLOCK, d), lambda i: (i, 0))],
        out_specs=pl.BlockSpec((ROWS_PER_BLOCK, d), lambda i: (i, 0)),
        out_shape=jax.ShapeDtypeStruct((n, d), x.dtype),
    )(x)
    sizes = jnp.array([n], dtype=jnp.int32)
    return (gathered, sizes)


# speedup vs baseline: 1.1064x; 1.1064x over previous
import jax
import jax.numpy as jnp
from jax.experimental import pallas as pl


ROWS_PER_BLOCK = 2048


def _copy_block(x_ref, o_ref):
    o_ref[...] = x_ref[...]


def kernel(x):
    n, d = x.shape
    grid = (n // ROWS_PER_BLOCK,)
    gathered = pl.pallas_call(
        _copy_block,
        grid=grid,
        in_specs=[pl.BlockSpec((ROWS_PER_BLOCK, d), lambda i: (i, 0))],
        out_specs=pl.BlockSpec((ROWS_PER_BLOCK, d), lambda i: (i, 0)),
        out_shape=jax.ShapeDtypeStruct((n, d), x.dtype),
    )(x)
    sizes = jnp.array([n], dtype=jnp.int32)
    return (gathered, sizes)
